# Initial kernel scaffold; baseline (speedup 1.0000x reference)
import jax
import jax.numpy as jnp
from jax.experimental import pallas as pl


def kernel(edge_index, h_src, h_dst, W, b):
    raise NotImplementedError("write your pallas kernel here")


# speedup vs baseline: 6.1809x; 6.1809x over previous
import functools

import jax
import jax.numpy as jnp
from jax import lax
from jax.experimental import pallas as pl
from jax.experimental.pallas import tpu as pltpu
from jax.experimental.pallas import tpu_sc as plsc

N = 10000
E = 320000
D = 128
OUT = 128

NC = 2
NS = 16
NW = NC * NS
EPT = E // NW
CHUNK = 80
NCHUNK = EPT // CHUNK
NPAD = 10240
RPT = NPAD // NS
ZBLK = 64
L = 16

_mesh = plsc.VectorSubcoreMesh(core_axis_name="c", subcore_axis_name="s")


@functools.partial(
    pl.kernel,
    out_type=(
        jax.ShapeDtypeStruct((NC * NPAD, D), jnp.float32),
        jax.ShapeDtypeStruct((NW, NPAD), jnp.float32),
    ),
    mesh=_mesh,
    compiler_params=pltpu.CompilerParams(needs_layout_passes=False),
    scratch_types=(
        pltpu.VMEM_SHARED((NPAD, D), jnp.float32),
        pltpu.VMEM((NPAD,), jnp.float32),
        pltpu.VMEM((ZBLK, D), jnp.float32),
        pltpu.VMEM((CHUNK,), jnp.int32),
        pltpu.VMEM((CHUNK,), jnp.int32),
        pltpu.VMEM((CHUNK, D), jnp.float32),
        pltpu.SemaphoreType.DMA,
    ),
)
def _sc_segment_sum(src_hbm, dst_hbm, hsrc_hbm, zf_hbm,
                    feats_out, counts_out,
                    feats_sp, cnt_v, zf_v, sidx, didx, rows_v, gsem):
    cid = lax.axis_index("c")
    sid = lax.axis_index("s")
    wid = cid * NS + sid

    pltpu.sync_copy(zf_hbm, zf_v)
    row0 = sid * RPT
    for k in range(RPT // ZBLK):
        pltpu.sync_copy(zf_v, feats_sp.at[pl.ds(row0 + k * ZBLK, ZBLK)])

    @pl.loop(0, NPAD // L)
    def zero_cnt(i):
        cnt_v[pl.ds(i * L, L)] = jnp.zeros((L,), jnp.float32)

    plsc.subcore_barrier()

    ebase = wid * EPT
    ones16 = jnp.ones((L,), jnp.float32)

    @pl.loop(0, NCHUNK)
    def step(i):
        base = ebase + i * CHUNK
        pltpu.sync_copy(src_hbm.at[pl.ds(base, CHUNK)], sidx)
        pltpu.sync_copy(dst_hbm.at[pl.ds(base, CHUNK)], didx)
        pltpu.async_copy(hsrc_hbm.at[sidx], rows_v, gsem).wait()
        pltpu.sync_copy(rows_v, feats_sp.at[didx], add=True)
        for j in range(CHUNK // L):
            idx16 = didx[pl.ds(j * L, L)]
            plsc.addupdate_scatter(cnt_v, [idx16], ones16)

    plsc.subcore_barrier()

    obase = cid * NPAD + row0
    for k in range(RPT // ZBLK):
        pltpu.sync_copy(feats_sp.at[pl.ds(row0 + k * ZBLK, ZBLK)], zf_v)
        pltpu.sync_copy(zf_v, feats_out.at[pl.ds(obase + k * ZBLK, ZBLK)])
    pltpu.sync_copy(cnt_v, counts_out.at[wid])


ROWS_BLK = 512
GRID = NPAD // ROWS_BLK


def _tc_body(f_ref, c_ref, hd_ref, w_ref, b_ref, o_ref):
    s = f_ref[0] + f_ref[1]
    cnt = jnp.sum(c_ref[...], axis=0)[:, None]
    h_n = s / jnp.maximum(cnt, 1.0)
    w_self = w_ref[:, :D]
    w_neigh = w_ref[:, D:]
    o = lax.dot_general(hd_ref[...], w_self, (((1,), (1,)), ((), ())),
                        preferred_element_type=jnp.float32)
    o = o + lax.dot_general(h_n, w_neigh, (((1,), (1,)), ((), ())),
                            preferred_element_type=jnp.float32)
    o_ref[...] = o + b_ref[...]


def kernel(edge_index, h_src, h_dst, W, b):
    src = edge_index[0]
    dst = edge_index[1]
    zf = jnp.zeros((ZBLK, D), jnp.float32)

    feats, counts = _sc_segment_sum(src, dst, h_src, zf)

    hd_pad = jnp.concatenate(
        [h_dst, jnp.zeros((NPAD - N, D), h_dst.dtype)], axis=0)

    out = pl.pallas_call(
        _tc_body,
        grid=(GRID,),
        in_specs=[
            pl.BlockSpec((NC, ROWS_BLK, D), lambda i: (0, i, 0)),
            pl.BlockSpec((NW, ROWS_BLK), lambda i: (0, i)),
            pl.BlockSpec((ROWS_BLK, D), lambda i: (i, 0)),
            pl.BlockSpec((OUT, 2 * D), lambda i: (0, 0)),
            pl.BlockSpec((1, OUT), lambda i: (0, 0)),
        ],
        out_specs=pl.BlockSpec((ROWS_BLK, OUT), lambda i: (i, 0)),
        out_shape=jax.ShapeDtypeStruct((NPAD, OUT), jnp.float32),
    )(feats.reshape(NC, NPAD, D), counts, hd_pad, W, b.reshape(1, OUT))
    return out[:N]


# speedup vs baseline: 7.1225x; 1.1523x over previous
import functools

import jax
import jax.numpy as jnp
from jax import lax
from jax.experimental import pallas as pl
from jax.experimental.pallas import tpu as pltpu
from jax.experimental.pallas import tpu_sc as plsc

N = 10000
E = 320000
D = 128
OUT = 128

NC = 2
NS = 16
NW = NC * NS
EPT = E // NW
CHUNK = 80
NCHUNK = EPT // CHUNK
NPAD = 10240
RPT = NPAD // NS
ZBLK = 64
L = 16

_mesh = plsc.VectorSubcoreMesh(core_axis_name="c", subcore_axis_name="s")


@functools.partial(
    pl.kernel,
    out_type=(
        jax.ShapeDtypeStruct((NC * NPAD, D), jnp.float32),
        jax.ShapeDtypeStruct((NW, NPAD), jnp.float32),
    ),
    mesh=_mesh,
    compiler_params=pltpu.CompilerParams(needs_layout_passes=False),
    scratch_types=(
        pltpu.VMEM_SHARED((NPAD, D), jnp.float32),
        pltpu.VMEM((NPAD,), jnp.float32),
        pltpu.VMEM((ZBLK, D), jnp.float32),
        tuple(pltpu.VMEM((CHUNK,), jnp.int32) for _ in range(4)),
        tuple(pltpu.VMEM((CHUNK,), jnp.int32) for _ in range(4)),
        tuple(pltpu.VMEM((CHUNK, D), jnp.float32) for _ in range(2)),
        tuple(pltpu.SemaphoreType.DMA for _ in range(2)),
        tuple(pltpu.SemaphoreType.DMA for _ in range(2)),
    ),
)
def _sc_segment_sum(src_hbm, dst_hbm, hsrc_hbm, zf_hbm,
                    feats_out, counts_out,
                    feats_sp, cnt_v, zf_v, sidx, didx, rows, gsem, ssem):
    cid = lax.axis_index("c")
    sid = lax.axis_index("s")
    wid = cid * NS + sid

    pltpu.sync_copy(zf_hbm, zf_v)
    row0 = sid * RPT
    for k in range(RPT // ZBLK):
        pltpu.sync_copy(zf_v, feats_sp.at[pl.ds(row0 + k * ZBLK, ZBLK)])

    @pl.loop(0, NPAD // L)
    def zero_cnt(i):
        cnt_v[pl.ds(i * L, L)] = jnp.zeros((L,), jnp.float32)

    plsc.subcore_barrier()

    ebase = wid * EPT
    ones16 = jnp.ones((L,), jnp.float32)

    def load_idx(c, ring):
        base = jnp.minimum(ebase + c * CHUNK, E - CHUNK)
        pltpu.sync_copy(src_hbm.at[pl.ds(base, CHUNK)], sidx[ring])
        pltpu.sync_copy(dst_hbm.at[pl.ds(base, CHUNK)], didx[ring])

    def bump_counts(ring):
        for j in range(CHUNK // L):
            plsc.addupdate_scatter(cnt_v, [didx[ring][pl.ds(j * L, L)]], ones16)

    load_idx(0, 0)
    load_idx(1, 1)
    load_idx(2, 2)
    pltpu.async_copy(hsrc_hbm.at[sidx[0]], rows[0], gsem[0]).wait()
    pltpu.async_copy(rows[0], feats_sp.at[didx[0]], ssem[0], add=True)
    bump_counts(0)
    load_idx(3, 3)
    pltpu.async_copy(hsrc_hbm.at[sidx[1]], rows[1], gsem[1])

    @pl.loop(0, (NCHUNK - 1) // 4)
    def step(i):
        for u in range(4):
            c = 1 + i * 4 + u
            s = (1 + u) & 1
            o = 1 - s
            rc = (1 + u) % 4
            rn1 = (2 + u) % 4
            rn2 = (3 + u) % 4
            pltpu.make_async_copy(hsrc_hbm.at[sidx[s]], rows[s],
                                  gsem[s]).wait()
            pltpu.async_copy(rows[s], feats_sp.at[didx[rc]], ssem[s],
                             add=True)
            bump_counts(rc)
            load_idx(c + 2, rn2)
            pltpu.make_async_copy(rows[o], feats_sp.at[didx[rc]],
                                  ssem[o]).wait()

            @pl.when(c < NCHUNK - 1)
            def _():
                pltpu.async_copy(hsrc_hbm.at[sidx[rn1]], rows[o], gsem[o])

    pltpu.make_async_copy(rows[0], feats_sp.at[didx[0]], ssem[0]).wait()
    plsc.subcore_barrier()

    obase = cid * NPAD + row0
    for k in range(RPT // ZBLK):
        pltpu.sync_copy(feats_sp.at[pl.ds(row0 + k * ZBLK, ZBLK)], zf_v)
        pltpu.sync_copy(zf_v, feats_out.at[pl.ds(obase + k * ZBLK, ZBLK)])
    pltpu.sync_copy(cnt_v, counts_out.at[wid])


ROWS_BLK = 512
GRID = NPAD // ROWS_BLK


def _tc_body(f_ref, c_ref, hd_ref, w_ref, b_ref, o_ref):
    s = f_ref[0] + f_ref[1]
    cnt = jnp.sum(c_ref[...], axis=0)[:, None]
    h_n = s / jnp.maximum(cnt, 1.0)
    w_self = w_ref[:, :D]
    w_neigh = w_ref[:, D:]
    o = lax.dot_general(hd_ref[...], w_self, (((1,), (1,)), ((), ())),
                        preferred_element_type=jnp.float32)
    o = o + lax.dot_general(h_n, w_neigh, (((1,), (1,)), ((), ())),
                            preferred_element_type=jnp.float32)
    o_ref[...] = o + b_ref[...]


def kernel(edge_index, h_src, h_dst, W, b):
    src = edge_index[0]
    dst = edge_index[1]
    zf = jnp.zeros((ZBLK, D), jnp.float32)

    feats, counts = _sc_segment_sum(src, dst, h_src, zf)

    hd_pad = jnp.concatenate(
        [h_dst, jnp.zeros((NPAD - N, D), h_dst.dtype)], axis=0)

    out = pl.pallas_call(
        _tc_body,
        grid=(GRID,),
        in_specs=[
            pl.BlockSpec((NC, ROWS_BLK, D), lambda i: (0, i, 0)),
            pl.BlockSpec((NW, ROWS_BLK), lambda i: (0, i)),
            pl.BlockSpec((ROWS_BLK, D), lambda i: (i, 0)),
            pl.BlockSpec((OUT, 2 * D), lambda i: (0, 0)),
            pl.BlockSpec((1, OUT), lambda i: (0, 0)),
        ],
        out_specs=pl.BlockSpec((ROWS_BLK, OUT), lambda i: (i, 0)),
        out_shape=jax.ShapeDtypeStruct((NPAD, OUT), jnp.float32),
    )(feats.reshape(NC, NPAD, D), counts, hd_pad, W, b.reshape(1, OUT))
    return out[:N]


# speedup vs baseline: 7.2281x; 1.0148x over previous
import functools

import jax
import jax.numpy as jnp
from jax import lax
from jax.experimental import pallas as pl
from jax.experimental.pallas import tpu as pltpu
from jax.experimental.pallas import tpu_sc as plsc

N = 10000
E = 320000
D = 128
OUT = 128

NC = 2
NS = 16
NW = NC * NS
EPT = E // NW
CHUNK = 80
NCHUNK = EPT // CHUNK
NPAD = 10240
RPT = NPAD // NS
ZBLK = 64
L = 16

_mesh = plsc.VectorSubcoreMesh(core_axis_name="c", subcore_axis_name="s")


@functools.partial(
    pl.kernel,
    out_type=(
        jax.ShapeDtypeStruct((NC * NPAD, D), jnp.float32),
        jax.ShapeDtypeStruct((NW, NPAD), jnp.float32),
    ),
    mesh=_mesh,
    compiler_params=pltpu.CompilerParams(needs_layout_passes=False),
    scratch_types=(
        pltpu.VMEM_SHARED((NPAD, D), jnp.float32),
        pltpu.VMEM((NPAD,), jnp.float32),
        pltpu.VMEM((ZBLK, D), jnp.float32),
        tuple(pltpu.VMEM((CHUNK,), jnp.int32) for _ in range(4)),
        tuple(pltpu.VMEM((CHUNK,), jnp.int32) for _ in range(4)),
        tuple(pltpu.VMEM((CHUNK, D), jnp.float32) for _ in range(2)),
        tuple(pltpu.SemaphoreType.DMA for _ in range(2)),
        tuple(pltpu.SemaphoreType.DMA for _ in range(2)),
    ),
)
def _sc_segment_sum(src_hbm, dst_hbm, hsrc_hbm, zf_hbm,
                    feats_out, counts_out,
                    feats_sp, cnt_v, zf_v, sidx, didx, rows, gsem, ssem):
    cid = lax.axis_index("c")
    sid = lax.axis_index("s")
    wid = cid * NS + sid

    pltpu.sync_copy(zf_hbm, zf_v)
    row0 = sid * RPT
    for k in range(RPT // ZBLK):
        pltpu.sync_copy(zf_v, feats_sp.at[pl.ds(row0 + k * ZBLK, ZBLK)])

    @pl.loop(0, NPAD // L)
    def zero_cnt(i):
        cnt_v[pl.ds(i * L, L)] = jnp.zeros((L,), jnp.float32)

    plsc.subcore_barrier()

    ebase = wid * EPT
    ones16 = jnp.ones((L,), jnp.float32)

    def load_idx(c, ring):
        base = jnp.minimum(ebase + c * CHUNK, E - CHUNK)
        pltpu.sync_copy(src_hbm.at[pl.ds(base, CHUNK)], sidx[ring])
        pltpu.sync_copy(dst_hbm.at[pl.ds(base, CHUNK)], didx[ring])

    def bump_counts(ring):
        for j in range(CHUNK // L):
            plsc.addupdate_scatter(cnt_v, [didx[ring][pl.ds(j * L, L)]], ones16)

    load_idx(0, 0)
    load_idx(1, 1)
    load_idx(2, 2)
    pltpu.async_copy(hsrc_hbm.at[sidx[0]], rows[0], gsem[0]).wait()
    load_idx(3, 3)
    pltpu.async_copy(hsrc_hbm.at[sidx[1]], rows[1], gsem[1])

    @pl.loop(0, (NCHUNK - 1) // 4)
    def step(i):
        for u in range(4):
            c = 1 + i * 4 + u
            s = (1 + u) & 1
            o = 1 - s
            rc = (1 + u) % 4
            rn1 = (2 + u) % 4
            rn2 = (3 + u) % 4
            pltpu.make_async_copy(hsrc_hbm.at[sidx[s]], rows[s],
                                  gsem[s]).wait()
            load_idx(c + 2, rn2)

            @pl.when(c < NCHUNK - 1)
            def _():
                pltpu.async_copy(hsrc_hbm.at[sidx[rn1]], rows[o], gsem[o])

    plsc.subcore_barrier()

    obase = cid * NPAD + row0
    for k in range(RPT // ZBLK):
        pltpu.sync_copy(feats_sp.at[pl.ds(row0 + k * ZBLK, ZBLK)], zf_v)
        pltpu.sync_copy(zf_v, feats_out.at[pl.ds(obase + k * ZBLK, ZBLK)])
    pltpu.sync_copy(cnt_v, counts_out.at[wid])


ROWS_BLK = 512
GRID = NPAD // ROWS_BLK


def _tc_body(f_ref, c_ref, hd_ref, w_ref, b_ref, o_ref):
    s = f_ref[0] + f_ref[1]
    cnt = jnp.sum(c_ref[...], axis=0)[:, None]
    h_n = s / jnp.maximum(cnt, 1.0)
    w_self = w_ref[:, :D]
    w_neigh = w_ref[:, D:]
    o = lax.dot_general(hd_ref[...], w_self, (((1,), (1,)), ((), ())),
                        preferred_element_type=jnp.float32)
    o = o + lax.dot_general(h_n, w_neigh, (((1,), (1,)), ((), ())),
                            preferred_element_type=jnp.float32)
    o_ref[...] = o + b_ref[...]


def kernel(edge_index, h_src, h_dst, W, b):
    src = edge_index[0]
    dst = edge_index[1]
    zf = jnp.zeros((ZBLK, D), jnp.float32)

    feats, counts = _sc_segment_sum(src, dst, h_src, zf)

    hd_pad = jnp.concatenate(
        [h_dst, jnp.zeros((NPAD - N, D), h_dst.dtype)], axis=0)

    out = pl.pallas_call(
        _tc_body,
        grid=(GRID,),
        in_specs=[
            pl.BlockSpec((NC, ROWS_BLK, D), lambda i: (0, i, 0)),
            pl.BlockSpec((NW, ROWS_BLK), lambda i: (0, i)),
            pl.BlockSpec((ROWS_BLK, D), lambda i: (i, 0)),
            pl.BlockSpec((OUT, 2 * D), lambda i: (0, 0)),
            pl.BlockSpec((1, OUT), lambda i: (0, 0)),
        ],
        out_specs=pl.BlockSpec((ROWS_BLK, OUT), lambda i: (i, 0)),
        out_shape=jax.ShapeDtypeStruct((NPAD, OUT), jnp.float32),
    )(feats.reshape(NC, NPAD, D), counts, hd_pad, W, b.reshape(1, OUT))
    return out[:N]


# speedup vs baseline: 10.8860x; 1.5061x over previous
import functools

import jax
import jax.numpy as jnp
from jax import lax
from jax.experimental import pallas as pl
from jax.experimental.pallas import tpu as pltpu
from jax.experimental.pallas import tpu_sc as plsc

N = 10000
E = 320000
D = 128
OUT = 128

NC = 2
NS = 16
NW = NC * NS
EPT = E // NW
CHUNK = 80
NCHUNK = EPT // CHUNK
NPAD = 10240
RPT = NPAD // NS
ZBLK = 32
L = 16

_mesh = plsc.VectorSubcoreMesh(core_axis_name="c", subcore_axis_name="s")


@functools.partial(
    pl.kernel,
    out_type=(
        jax.ShapeDtypeStruct((NC * NPAD, D), jnp.float32),
        jax.ShapeDtypeStruct((NW, NPAD), jnp.float32),
    ),
    mesh=_mesh,
    compiler_params=pltpu.CompilerParams(needs_layout_passes=False),
    scratch_types=(
        pltpu.VMEM_SHARED((NPAD, D), jnp.float32),
        pltpu.VMEM((NPAD,), jnp.float32),
        pltpu.VMEM((ZBLK, D), jnp.float32),
        tuple(pltpu.VMEM((CHUNK,), jnp.int32) for _ in range(3)),
        tuple(pltpu.VMEM((CHUNK,), jnp.int32) for _ in range(3)),
        tuple(pltpu.VMEM((CHUNK, D), jnp.float32) for _ in range(3)),
        tuple(pltpu.SemaphoreType.DMA for _ in range(3)),
        tuple(pltpu.SemaphoreType.DMA for _ in range(3)),
    ),
)
def _sc_segment_sum(src_hbm, dst_hbm, hsrc_hbm, zf_hbm,
                    feats_out, counts_out,
                    feats_sp, cnt_v, zf_v, sidx, didx, rows, gsem, ssem):
    cid = lax.axis_index("c")
    sid = lax.axis_index("s")
    wid = cid * NS + sid

    pltpu.sync_copy(zf_hbm, zf_v)
    row0 = sid * RPT
    for k in range(RPT // ZBLK):
        pltpu.sync_copy(zf_v, feats_sp.at[pl.ds(row0 + k * ZBLK, ZBLK)])

    @pl.loop(0, NPAD // L)
    def zero_cnt(i):
        cnt_v[pl.ds(i * L, L)] = jnp.zeros((L,), jnp.float32)

    plsc.subcore_barrier()

    ebase = wid * EPT
    ones16 = jnp.ones((L,), jnp.float32)

    def load_idx(c, ring):
        base = jnp.minimum(ebase + c * CHUNK, E - CHUNK)
        pltpu.sync_copy(src_hbm.at[pl.ds(base, CHUNK)], sidx[ring])
        pltpu.sync_copy(dst_hbm.at[pl.ds(base, CHUNK)], didx[ring])

    def bump_counts(ring):
        for j in range(CHUNK // L):
            plsc.addupdate_scatter(cnt_v, [didx[ring][pl.ds(j * L, L)]], ones16)

    load_idx(0, 0)
    load_idx(1, 1)
    pltpu.async_copy(hsrc_hbm.at[sidx[0]], rows[0], gsem[0])
    pltpu.async_copy(hsrc_hbm.at[sidx[1]], rows[1], gsem[1])
    pltpu.make_async_copy(hsrc_hbm.at[sidx[0]], rows[0], gsem[0]).wait()
    pltpu.async_copy(rows[0], feats_sp.at[didx[0]], ssem[0], add=True)
    bump_counts(0)
    load_idx(2, 2)
    pltpu.async_copy(hsrc_hbm.at[sidx[2]], rows[2], gsem[2])

    @pl.loop(0, (NCHUNK - 2) // 3)
    def step(i):
        for u in range(3):
            c = 1 + i * 3 + u
            r = (1 + u) % 3
            rp = u
            pltpu.make_async_copy(hsrc_hbm.at[sidx[r]], rows[r],
                                  gsem[r]).wait()
            pltpu.async_copy(rows[r], feats_sp.at[didx[r]], ssem[r],
                             add=True)
            bump_counts(r)
            pltpu.make_async_copy(rows[rp], feats_sp.at[didx[rp]],
                                  ssem[rp]).wait()
            load_idx(c + 2, rp)

            @pl.when(c < NCHUNK - 2)
            def _():
                pltpu.async_copy(hsrc_hbm.at[sidx[rp]], rows[rp], gsem[rp])

    pltpu.make_async_copy(hsrc_hbm.at[sidx[1]], rows[1], gsem[1]).wait()
    pltpu.async_copy(rows[1], feats_sp.at[didx[1]], ssem[1], add=True)
    bump_counts(1)
    pltpu.make_async_copy(rows[0], feats_sp.at[didx[0]], ssem[0]).wait()
    pltpu.make_async_copy(rows[1], feats_sp.at[didx[1]], ssem[1]).wait()
    plsc.subcore_barrier()

    obase = cid * NPAD + row0
    for k in range(RPT // ZBLK):
        pltpu.sync_copy(feats_sp.at[pl.ds(row0 + k * ZBLK, ZBLK)], zf_v)
        pltpu.sync_copy(zf_v, feats_out.at[pl.ds(obase + k * ZBLK, ZBLK)])
    pltpu.sync_copy(cnt_v, counts_out.at[wid])


ROWS_BLK = 512
GRID = NPAD // ROWS_BLK


def _tc_body(f_ref, c_ref, hd_ref, w_ref, b_ref, o_ref):
    s = f_ref[0] + f_ref[1]
    cnt = jnp.sum(c_ref[...], axis=0)[:, None]
    h_n = s / jnp.maximum(cnt, 1.0)
    w_self = w_ref[:, :D]
    w_neigh = w_ref[:, D:]
    o = lax.dot_general(hd_ref[...], w_self, (((1,), (1,)), ((), ())),
                        preferred_element_type=jnp.float32)
    o = o + lax.dot_general(h_n, w_neigh, (((1,), (1,)), ((), ())),
                            preferred_element_type=jnp.float32)
    o_ref[...] = o + b_ref[...]


def kernel(edge_index, h_src, h_dst, W, b):
    src = edge_index[0]
    dst = edge_index[1]
    zf = jnp.zeros((ZBLK, D), jnp.float32)

    feats, counts = _sc_segment_sum(src, dst, h_src, zf)

    hd_pad = jnp.concatenate(
        [h_dst, jnp.zeros((NPAD - N, D), h_dst.dtype)], axis=0)

    out = pl.pallas_call(
        _tc_body,
        grid=(GRID,),
        in_specs=[
            pl.BlockSpec((NC, ROWS_BLK, D), lambda i: (0, i, 0)),
            pl.BlockSpec((NW, ROWS_BLK), lambda i: (0, i)),
            pl.BlockSpec((ROWS_BLK, D), lambda i: (i, 0)),
            pl.BlockSpec((OUT, 2 * D), lambda i: (0, 0)),
            pl.BlockSpec((1, OUT), lambda i: (0, 0)),
        ],
        out_specs=pl.BlockSpec((ROWS_BLK, OUT), lambda i: (i, 0)),
        out_shape=jax.ShapeDtypeStruct((NPAD, OUT), jnp.float32),
    )(feats.reshape(NC, NPAD, D), counts, hd_pad, W, b.reshape(1, OUT))
    return out[:N]


# speedup vs baseline: 11.6558x; 1.0707x over previous
import functools

import jax
import jax.numpy as jnp
from jax import lax
from jax.experimental import pallas as pl
from jax.experimental.pallas import tpu as pltpu
from jax.experimental.pallas import tpu_sc as plsc

N = 10000
E = 320000
D = 128
OUT = 128

NC = 2
NS = 16
NW = NC * NS
EPT = E // NW
CHUNK = 80
NCHUNK = EPT // CHUNK
NPAD = 10240
RPT = NPAD // NS
ZBLK = 32
L = 16

_mesh = plsc.VectorSubcoreMesh(core_axis_name="c", subcore_axis_name="s")


@functools.partial(
    pl.kernel,
    out_type=(
        jax.ShapeDtypeStruct((NC * NPAD, D), jnp.float32),
        jax.ShapeDtypeStruct((NW, NPAD), jnp.float32),
    ),
    mesh=_mesh,
    compiler_params=pltpu.CompilerParams(needs_layout_passes=False),
    scratch_types=(
        pltpu.VMEM_SHARED((NPAD, D), jnp.float32),
        pltpu.VMEM((NPAD,), jnp.float32),
        pltpu.VMEM((ZBLK, D), jnp.float32),
        tuple(pltpu.VMEM((CHUNK,), jnp.int32) for _ in range(3)),
        tuple(pltpu.VMEM((CHUNK,), jnp.int32) for _ in range(3)),
        tuple(pltpu.VMEM((CHUNK, D), jnp.float32) for _ in range(3)),
        tuple(pltpu.SemaphoreType.DMA for _ in range(3)),
        tuple(pltpu.SemaphoreType.DMA for _ in range(3)),
        tuple(pltpu.SemaphoreType.DMA for _ in range(3)),
    ),
)
def _sc_segment_sum(src_hbm, dst_hbm, hsrc_hbm, zf_hbm,
                    feats_out, counts_out,
                    feats_sp, cnt_v, zf_v, sidx, didx, rows, gsem, ssem,
                    isem):
    cid = lax.axis_index("c")
    sid = lax.axis_index("s")
    wid = cid * NS + sid

    pltpu.sync_copy(zf_hbm, zf_v)
    row0 = sid * RPT
    for k in range(RPT // ZBLK):
        pltpu.sync_copy(zf_v, feats_sp.at[pl.ds(row0 + k * ZBLK, ZBLK)])

    @pl.loop(0, NPAD // L)
    def zero_cnt(i):
        cnt_v[pl.ds(i * L, L)] = jnp.zeros((L,), jnp.float32)

    plsc.subcore_barrier()

    ebase = wid * EPT
    ones16 = jnp.ones((L,), jnp.float32)

    def clamped_base(c):
        return jnp.minimum(ebase + c * CHUNK, E - CHUNK)

    def load_idx(c, ring):
        base = clamped_base(c)
        pltpu.sync_copy(src_hbm.at[pl.ds(base, CHUNK)], sidx[ring])
        pltpu.sync_copy(dst_hbm.at[pl.ds(base, CHUNK)], didx[ring])

    def bump_counts(ring):
        for j in range(CHUNK // L):
            plsc.addupdate_scatter(cnt_v, [didx[ring][pl.ds(j * L, L)]], ones16)

    load_idx(0, 0)
    load_idx(1, 1)
    pltpu.async_copy(hsrc_hbm.at[sidx[0]], rows[0], gsem[0])
    pltpu.async_copy(hsrc_hbm.at[sidx[1]], rows[1], gsem[1])
    pltpu.make_async_copy(hsrc_hbm.at[sidx[0]], rows[0], gsem[0]).wait()
    pltpu.async_copy(src_hbm.at[pl.ds(clamped_base(2), CHUNK)], sidx[2],
                     isem[2])
    pltpu.async_copy(rows[0], feats_sp.at[didx[0]], ssem[0], add=True)
    bump_counts(0)
    pltpu.make_async_copy(src_hbm.at[pl.ds(0, CHUNK)], sidx[2],
                          isem[2]).wait()
    pltpu.async_copy(hsrc_hbm.at[sidx[2]], rows[2], gsem[2])
    pltpu.sync_copy(dst_hbm.at[pl.ds(clamped_base(2), CHUNK)], didx[2])

    @pl.loop(0, (NCHUNK - 2) // 3)
    def step(i):
        for u in range(3):
            c = 1 + i * 3 + u
            r = (1 + u) % 3
            rp = u
            pltpu.make_async_copy(hsrc_hbm.at[sidx[r]], rows[r],
                                  gsem[r]).wait()
            pltpu.async_copy(src_hbm.at[pl.ds(clamped_base(c + 2), CHUNK)],
                             sidx[rp], isem[rp])
            pltpu.async_copy(rows[r], feats_sp.at[didx[r]], ssem[r],
                             add=True)
            bump_counts(r)
            pltpu.make_async_copy(rows[rp], feats_sp.at[didx[rp]],
                                  ssem[rp]).wait()
            pltpu.make_async_copy(src_hbm.at[pl.ds(0, CHUNK)], sidx[rp],
                                  isem[rp]).wait()

            @pl.when(c < NCHUNK - 2)
            def _():
                pltpu.async_copy(hsrc_hbm.at[sidx[rp]], rows[rp], gsem[rp])

            pltpu.sync_copy(dst_hbm.at[pl.ds(clamped_base(c + 2), CHUNK)],
                            didx[rp])

    pltpu.make_async_copy(hsrc_hbm.at[sidx[1]], rows[1], gsem[1]).wait()
    pltpu.async_copy(rows[1], feats_sp.at[didx[1]], ssem[1], add=True)
    bump_counts(1)
    pltpu.make_async_copy(rows[0], feats_sp.at[didx[0]], ssem[0]).wait()
    pltpu.make_async_copy(rows[1], feats_sp.at[didx[1]], ssem[1]).wait()
    plsc.subcore_barrier()

    obase = cid * NPAD + row0
    for k in range(RPT // ZBLK):
        pltpu.sync_copy(feats_sp.at[pl.ds(row0 + k * ZBLK, ZBLK)], zf_v)
        pltpu.sync_copy(zf_v, feats_out.at[pl.ds(obase + k * ZBLK, ZBLK)])
    pltpu.sync_copy(cnt_v, counts_out.at[wid])


ROWS_BLK = 512
GRID = NPAD // ROWS_BLK


def _tc_body(f_ref, c_ref, hd_ref, w_ref, b_ref, o_ref):
    s = f_ref[0] + f_ref[1]
    cnt = jnp.sum(c_ref[...], axis=0)[:, None]
    h_n = s / jnp.maximum(cnt, 1.0)
    w_self = w_ref[:, :D]
    w_neigh = w_ref[:, D:]
    o = lax.dot_general(hd_ref[...], w_self, (((1,), (1,)), ((), ())),
                        preferred_element_type=jnp.float32)
    o = o + lax.dot_general(h_n, w_neigh, (((1,), (1,)), ((), ())),
                            preferred_element_type=jnp.float32)
    o_ref[...] = o + b_ref[...]


def kernel(edge_index, h_src, h_dst, W, b):
    src = edge_index[0]
    dst = edge_index[1]
    zf = jnp.zeros((ZBLK, D), jnp.float32)

    feats, counts = _sc_segment_sum(src, dst, h_src, zf)

    hd_pad = jnp.concatenate(
        [h_dst, jnp.zeros((NPAD - N, D), h_dst.dtype)], axis=0)

    out = pl.pallas_call(
        _tc_body,
        grid=(GRID,),
        in_specs=[
            pl.BlockSpec((NC, ROWS_BLK, D), lambda i: (0, i, 0)),
            pl.BlockSpec((NW, ROWS_BLK), lambda i: (0, i)),
            pl.BlockSpec((ROWS_BLK, D), lambda i: (i, 0)),
            pl.BlockSpec((OUT, 2 * D), lambda i: (0, 0)),
            pl.BlockSpec((1, OUT), lambda i: (0, 0)),
        ],
        out_specs=pl.BlockSpec((ROWS_BLK, OUT), lambda i: (i, 0)),
        out_shape=jax.ShapeDtypeStruct((NPAD, OUT), jnp.float32),
    )(feats.reshape(NC, NPAD, D), counts, hd_pad, W, b.reshape(1, OUT))
    return out[:N]
